# baseline (speedup 1.0000x reference)
import functools

import numpy as np

import jax
import jax.numpy as jnp
from jax import lax
from jax.experimental import pallas as pl
from jax.experimental.pallas import tpu as pltpu
from jax.experimental.pallas import tpu_sc as plsc

N = 10000
E = 320000
D = 128
NC = 2
NS = 16
NW = NC * NS
EPT = E // NW
CH = 128
NCHUNK = 80
EPT_PAD = NCHUNK * CH
NPAD = 10240
CHS = 64
NCHS = 164
EPT_S = NCHS * CHS
NACC = 10240

_mesh = plsc.VectorSubcoreMesh(core_axis_name="c", subcore_axis_name="s")

_PERM = np.empty((D,), np.int32)
for _g in range(4):
    for _u in range(16):
        _PERM[32 * _g + 2 * _u] = 32 * _g + _u
        _PERM[32 * _g + 2 * _u + 1] = 32 * _g + 16 + _u
_sc_params = pltpu.CompilerParams(needs_layout_passes=False)
_sc_params_nt = pltpu.CompilerParams(needs_layout_passes=False,
                                     use_tc_tiling_on_sc=False)


COLS = NPAD // NS


@functools.partial(
    pl.kernel,
    out_type=jax.ShapeDtypeStruct((NC, 2, NPAD), jnp.float32),
    mesh=_mesh,
    compiler_params=_sc_params,
    scratch_types=[
        pltpu.VMEM((EPT_PAD,), jnp.int32),
        pltpu.VMEM((EPT_PAD,), jnp.int32),
        pltpu.VMEM((NPAD,), jnp.float32),
        pltpu.VMEM((NPAD,), jnp.float32),
        pltpu.VMEM((COLS,), jnp.float32),
        pltpu.VMEM((COLS,), jnp.float32),
        pltpu.VMEM_SHARED((NS, 2, NPAD), jnp.float32),
    ],
)
def _deg_kernel(src_hbm, dst_hbm, out_hbm, src_v, dst_v, p_out, p_in, tmp, accv, slab):
    c = lax.axis_index("c")
    s = lax.axis_index("s")
    w = c * NS + s
    pltpu.sync_copy(src_hbm.at[w], src_v)
    pltpu.sync_copy(dst_hbm.at[w], dst_v)

    zero16 = jnp.zeros((16,), jnp.float32)

    def _zero(i, _):
        p_out[pl.ds(i * 16, 16)] = zero16
        p_in[pl.ds(i * 16, 16)] = zero16
        return _
    lax.fori_loop(0, NPAD // 16, _zero, None)

    ones16 = jnp.ones((16,), jnp.float32)

    def _hist(i, _):
        si = src_v[pl.ds(i * 16, 16)]
        plsc.addupdate_scatter(p_out, [si], ones16)
        di = dst_v[pl.ds(i * 16, 16)]
        plsc.addupdate_scatter(p_in, [di], ones16)
        return _
    lax.fori_loop(0, EPT_PAD // 16, _hist, None)

    pltpu.sync_copy(p_out, slab.at[s, 0])
    pltpu.sync_copy(p_in, slab.at[s, 1])
    plsc.subcore_barrier()

    for k in (0, 1):
        def _zacc(i, _):
            accv[pl.ds(i * 16, 16)] = zero16
            return _
        lax.fori_loop(0, COLS // 16, _zacc, None)

        def _red(j, _):
            pltpu.sync_copy(slab.at[j, k, pl.ds(s * COLS, COLS)], tmp)

            def _add(i, _):
                accv[pl.ds(i * 16, 16)] += tmp[pl.ds(i * 16, 16)]
                return _
            lax.fori_loop(0, COLS // 16, _add, None)
            return _
        lax.fori_loop(0, NS, _red, None)
        pltpu.sync_copy(accv, out_hbm.at[c, k, pl.ds(s * COLS, COLS)])


DW = D // 2


@functools.partial(
    pl.kernel,
    out_type=jax.ShapeDtypeStruct((NC, NACC, D), jnp.float32),
    mesh=_mesh,
    compiler_params=_sc_params_nt,
    scratch_types=[
        pltpu.VMEM((4, CH, DW), jnp.int32),
        pltpu.VMEM((CH // 2, D), jnp.float32),
        pltpu.VMEM_SHARED((NACC, D), jnp.float32),
    ] + [pltpu.VMEM((CH,), jnp.int32)] * 4
      + [pltpu.VMEM((2, CH // 2), jnp.int32)] * 4
      + [pltpu.SemaphoreType.DMA] * 12,
)
def _spmm_kernel(h_hbm, src_hbm, dst_hbm, out_hbm, gbuf, fbuf, acc, *rest):
    sbuf = rest[0:4]
    dbuf = rest[4:8]
    gsem = rest[8:12]
    xsem = rest[12:16]
    dsem = rest[16:20]
    c = lax.axis_index("c")
    s = lax.axis_index("s")
    w = c * NS + s

    zero16 = jnp.zeros((16,), jnp.float32)

    def _zero(i, _):
        def _zrow(jj, _2):
            fbuf[i, pl.ds(jj * 16, 16)] = zero16
            return _2
        lax.fori_loop(0, D // 16, _zrow, None)
        return _
    lax.fori_loop(0, CH // 2, _zero, None)

    rows_per_tile = NACC // NS
    HCH = CH // 2

    def _zacc(k, _):
        pltpu.sync_copy(fbuf,
                        acc.at[pl.ds(s * rows_per_tile + k * HCH, HCH)])
        return _
    lax.fori_loop(0, rows_per_tile // HCH, _zacc, None)
    plsc.subcore_barrier()

    for b in range(4):
        pltpu.async_copy(src_hbm.at[w, b], sbuf[b], xsem[b])
        pltpu.async_copy(dst_hbm.at[w, b], dbuf[b], dsem[b])
    for b in range(3):
        pltpu.make_async_copy(src_hbm.at[w, b], sbuf[b], xsem[b]).wait()
        pltpu.async_copy(h_hbm.at[sbuf[b]], gbuf.at[b], gsem[b])

    mask_hi = jnp.full((16,), -65536, jnp.int32)

    def _convert_scatter(b, hh):
        def _row(r, _):
            def _grp(g, _2):
                x = gbuf[b, hh * HCH + r, pl.ds(g * 16, 16)]
                lo = plsc.bitcast(x << 16, jnp.float32)
                hi = plsc.bitcast(x & mask_hi, jnp.float32)
                fbuf[r, pl.ds(g * 32, 16)] = lo
                fbuf[r, pl.ds(g * 32 + 16, 16)] = hi
                return _2
            lax.fori_loop(0, DW // 16, _grp, None)
            return _
        lax.fori_loop(0, HCH, _row, None)
        pltpu.sync_copy(fbuf, acc.at[dbuf[b].at[hh]], add=True)

    def _step(i, _):
        for b in range(4):
            j = i * 4 + b
            b3 = (b + 3) % 4
            pltpu.make_async_copy(h_hbm.at[sbuf[b]], gbuf.at[b],
                                  gsem[b]).wait()

            @pl.when(j + 3 < NCHUNK)
            def _():
                pltpu.make_async_copy(src_hbm.at[w, j + 3], sbuf[b3],
                                      xsem[b3]).wait()
                pltpu.async_copy(h_hbm.at[sbuf[b3]], gbuf.at[b3], gsem[b3])

            pltpu.make_async_copy(dst_hbm.at[w, j], dbuf[b], dsem[b]).wait()
            _convert_scatter(b, 0)
            _convert_scatter(b, 1)

            @pl.when(j + 4 < NCHUNK)
            def _():
                pltpu.async_copy(src_hbm.at[w, j + 4], sbuf[b], xsem[b])
                pltpu.async_copy(dst_hbm.at[w, j + 4], dbuf[b], dsem[b])
        return _
    lax.fori_loop(0, NCHUNK // 4, _step, None)
    plsc.subcore_barrier()

    def _wb(k, _):
        r = s * rows_per_tile + k * HCH
        pltpu.sync_copy(acc.at[pl.ds(r, HCH)], fbuf)
        pltpu.sync_copy(fbuf, out_hbm.at[c, pl.ds(r, HCH)])
        return _
    lax.fori_loop(0, rows_per_tile // HCH, _wb, None)


def _scales(deg_blk):
    a_out = lax.rsqrt(jnp.clip(deg_blk[0, 0] + deg_blk[1, 0], 1.0, None))
    a_in = lax.rsqrt(jnp.clip(deg_blk[0, 1] + deg_blk[1, 1], 1.0, None))
    return a_out, a_in


def _tc_first_body(x_ref, deg_ref, w_ref, o_ref):
    a_out, _ = _scales(deg_ref)
    r = jnp.dot(x_ref[...], w_ref[...],
                preferred_element_type=jnp.float32) * a_out
    o_ref[...] = r.astype(jnp.bfloat16)


def _tc_mid_body(p_ref, deg_ref, b_ref, w_ref, o_ref):
    a_out, a_in = _scales(deg_ref)
    t = (p_ref[0] + p_ref[1]) * a_in + b_ref[...]
    t = jnp.maximum(t, 0.0)
    r = jnp.dot(t, w_ref[...],
                preferred_element_type=jnp.float32) * a_out
    o_ref[...] = r.astype(jnp.bfloat16)


def _tc_final_body(p_ref, deg_ref, b_ref, o_ref):
    _, a_in = _scales(deg_ref)
    o_ref[...] = (p_ref[0] + p_ref[1]) * a_in + b_ref[...]


BLK = 2000
GRID = N // BLK

_deg_spec = pl.BlockSpec((NC, 2, BLK, 1), lambda b: (0, 0, b, 0))
_p_spec = pl.BlockSpec((NC, BLK, D), lambda b: (0, b, 0))
_w_spec = pl.BlockSpec((D, D), lambda b: (0, 0))
_b_spec = pl.BlockSpec((1, D), lambda b: (0, 0))
_x_spec = pl.BlockSpec((BLK, D), lambda b: (b, 0))

_tc_first = pl.pallas_call(
    _tc_first_body,
    grid=(GRID,),
    in_specs=[_x_spec, _deg_spec, _w_spec],
    out_specs=_x_spec,
    out_shape=jax.ShapeDtypeStruct((N, D), jnp.bfloat16),
)

_tc_mid = pl.pallas_call(
    _tc_mid_body,
    grid=(GRID,),
    in_specs=[_p_spec, _deg_spec, _b_spec, _w_spec],
    out_specs=_x_spec,
    out_shape=jax.ShapeDtypeStruct((N, D), jnp.bfloat16),
)

_tc_final = pl.pallas_call(
    _tc_final_body,
    grid=(GRID,),
    in_specs=[_p_spec, _deg_spec, _b_spec],
    out_specs=_x_spec,
    out_shape=jax.ShapeDtypeStruct((N, D), jnp.float32),
)


def kernel(x, edge_index, W1, b1, W2, b2, W3, b3):
    src = edge_index[0].reshape(NW, EPT)
    dst = edge_index[1].reshape(NW, EPT)
    pad_n = jnp.full((NW, EPT_PAD - EPT), N, jnp.int32)
    pad_ns = jnp.full((NW, EPT_S - EPT), N, jnp.int32)
    pad_0s = jnp.zeros((NW, EPT_S - EPT), jnp.int32)
    srcd = jnp.concatenate([src, pad_n], axis=1)
    dstd = jnp.concatenate([dst, pad_n], axis=1)
    srcp = jnp.concatenate([src, jnp.zeros((NW, EPT_PAD - EPT), jnp.int32)],
                           axis=1).reshape(NW, NCHUNK, CH)
    dstp = jnp.concatenate([dst, jnp.full((NW, EPT_PAD - EPT), N, jnp.int32)],
                           axis=1).reshape(NW, NCHUNK, 2, CH // 2)

    deg = _deg_kernel(srcd, dstd)
    deg4 = deg.reshape(NC, 2, NPAD, 1)
    b1r, b2r, b3r = (b.reshape(1, D) for b in (b1, b2, b3))
    W1p, W2p, W3p = (W[:, _PERM] for W in (W1, W2, W3))

    def pack(h):
        return jax.lax.bitcast_convert_type(h.reshape(N, DW, 2), jnp.int32)

    h = _tc_first(x, deg4, W1p)
    p = _spmm_kernel(pack(h), srcp, dstp)
    h = _tc_mid(p, deg4, b1r, W2p)
    p = _spmm_kernel(pack(h), srcp, dstp)
    h = _tc_mid(p, deg4, b2r, W3p)
    p = _spmm_kernel(pack(h), srcp, dstp)
    return _tc_final(p, deg4, b3r)


# speedup vs baseline: 1.0922x; 1.0922x over previous
import functools

import jax
import jax.numpy as jnp
from jax import lax
from jax.experimental import pallas as pl
from jax.experimental.pallas import tpu as pltpu
from jax.experimental.pallas import tpu_sc as plsc

N = 10000
E = 320000
D = 128
NC = 2
NS = 16
NW = NC * NS
EPT = E // NW
CH = 128
NCHUNK = 80
EPT_PAD = NCHUNK * CH
NPAD = 10240
CHS = 64
NCHS = 164
EPT_S = NCHS * CHS
NACC = 10240

_mesh = plsc.VectorSubcoreMesh(core_axis_name="c", subcore_axis_name="s")
_sc_params = pltpu.CompilerParams(needs_layout_passes=False)


COLS = NPAD // NS


@functools.partial(
    pl.kernel,
    out_type=jax.ShapeDtypeStruct((NC, 2, NPAD), jnp.float32),
    mesh=_mesh,
    compiler_params=_sc_params,
    scratch_types=[
        pltpu.VMEM((EPT_PAD,), jnp.int32),
        pltpu.VMEM((EPT_PAD,), jnp.int32),
        pltpu.VMEM((NPAD,), jnp.float32),
        pltpu.VMEM((NPAD,), jnp.float32),
        pltpu.VMEM((COLS,), jnp.float32),
        pltpu.VMEM((COLS,), jnp.float32),
        pltpu.VMEM_SHARED((NS, 2, NPAD), jnp.float32),
    ],
)
def _deg_kernel(src_hbm, dst_hbm, out_hbm, src_v, dst_v, p_out, p_in, tmp, accv, slab):
    c = lax.axis_index("c")
    s = lax.axis_index("s")
    w = c * NS + s
    pltpu.sync_copy(src_hbm.at[w], src_v)
    pltpu.sync_copy(dst_hbm.at[w], dst_v)

    zero16 = jnp.zeros((16,), jnp.float32)

    def _zero(i, _):
        p_out[pl.ds(i * 16, 16)] = zero16
        p_in[pl.ds(i * 16, 16)] = zero16
        return _
    lax.fori_loop(0, NPAD // 16, _zero, None)

    ones16 = jnp.ones((16,), jnp.float32)

    def _hist(i, _):
        si = src_v[pl.ds(i * 16, 16)]
        plsc.addupdate_scatter(p_out, [si], ones16)
        di = dst_v[pl.ds(i * 16, 16)]
        plsc.addupdate_scatter(p_in, [di], ones16)
        return _
    lax.fori_loop(0, EPT_PAD // 16, _hist, None)

    pltpu.sync_copy(p_out, slab.at[s, 0])
    pltpu.sync_copy(p_in, slab.at[s, 1])
    plsc.subcore_barrier()

    for k in (0, 1):
        def _zacc(i, _):
            accv[pl.ds(i * 16, 16)] = zero16
            return _
        lax.fori_loop(0, COLS // 16, _zacc, None)

        def _red(j, _):
            pltpu.sync_copy(slab.at[j, k, pl.ds(s * COLS, COLS)], tmp)

            def _add(i, _):
                accv[pl.ds(i * 16, 16)] += tmp[pl.ds(i * 16, 16)]
                return _
            lax.fori_loop(0, COLS // 16, _add, None)
            return _
        lax.fori_loop(0, NS, _red, None)
        pltpu.sync_copy(accv, out_hbm.at[c, k, pl.ds(s * COLS, COLS)])


NB = 2
LAG = 1


@functools.partial(
    pl.kernel,
    out_type=jax.ShapeDtypeStruct((NC, NACC, D), jnp.float32),
    mesh=_mesh,
    compiler_params=_sc_params,
    scratch_types=[
        pltpu.VMEM((NCHUNK, CH), jnp.int32),
        pltpu.VMEM((NB, CH, D), jnp.float32),
        pltpu.VMEM_SHARED((NACC, D), jnp.float32),
    ] + [pltpu.VMEM((CH,), jnp.int32)] * NB
      + [pltpu.SemaphoreType.DMA] * (3 * NB),
)
def _spmm_kernel(h_hbm, src_hbm, dst_hbm, out_hbm, src_v, gbuf, acc, *rest):
    dbuf = rest[:NB]
    gsem = rest[NB:2 * NB]
    ssem = rest[2 * NB:3 * NB]
    dsem = rest[3 * NB:]
    c = lax.axis_index("c")
    s = lax.axis_index("s")
    w = c * NS + s
    pltpu.sync_copy(src_hbm.at[w], src_v)

    zero16 = jnp.zeros((16,), jnp.float32)

    def _zero(i, _):
        def _zrow(j, _2):
            gbuf[0, i, pl.ds(j * 16, 16)] = zero16
            return _2
        lax.fori_loop(0, D // 16, _zrow, None)
        return _
    lax.fori_loop(0, CH, _zero, None)

    rows_per_tile = NACC // NS

    def _zacc(k, _):
        pltpu.sync_copy(gbuf.at[0],
                        acc.at[pl.ds(s * rows_per_tile + k * CH, CH)])
        return _
    lax.fori_loop(0, rows_per_tile // CH, _zacc, None)
    plsc.subcore_barrier()

    for b in range(NB):
        pltpu.async_copy(h_hbm.at[src_v.at[b]], gbuf.at[b], gsem[b])
        pltpu.async_copy(dst_hbm.at[w, b], dbuf[b], dsem[b])

    def _step(i, _):
        for b in range(NB):
            j = i * NB + b
            pltpu.make_async_copy(h_hbm.at[src_v.at[j]], gbuf.at[b],
                                  gsem[b]).wait()
            pltpu.make_async_copy(dst_hbm.at[w, j], dbuf[b], dsem[b]).wait()
            pltpu.async_copy(gbuf.at[b], acc.at[dbuf[b]], ssem[b], add=True)

            g = j + NB - LAG
            bg = (b + NB - LAG) % NB

            @pl.when(jnp.logical_and(g >= NB, g < NCHUNK))
            def _():
                pltpu.make_async_copy(gbuf.at[bg], acc.at[dbuf[bg]],
                                      ssem[bg]).wait()
                pltpu.async_copy(h_hbm.at[src_v.at[g]], gbuf.at[bg], gsem[bg])
                pltpu.async_copy(dst_hbm.at[w, g], dbuf[bg], dsem[bg])
        return _
    lax.fori_loop(0, NCHUNK // NB, _step, None)

    for b in range(NB):
        pltpu.make_async_copy(gbuf.at[b], acc.at[dbuf[b]], ssem[b]).wait()
    plsc.subcore_barrier()

    def _wb(k, _):
        r = s * rows_per_tile + k * CH
        pltpu.sync_copy(acc.at[pl.ds(r, CH)], gbuf.at[0])
        pltpu.sync_copy(gbuf.at[0], out_hbm.at[c, pl.ds(r, CH)])
        return _
    lax.fori_loop(0, rows_per_tile // CH, _wb, None)


def _scales(deg_blk):
    a_out = lax.rsqrt(jnp.clip(deg_blk[0, 0] + deg_blk[1, 0], 1.0, None))
    a_in = lax.rsqrt(jnp.clip(deg_blk[0, 1] + deg_blk[1, 1], 1.0, None))
    return a_out, a_in


def _tc_first_body(x_ref, deg_ref, w_ref, o_ref):
    a_out, _ = _scales(deg_ref)
    o_ref[...] = jnp.dot(x_ref[...], w_ref[...],
                         preferred_element_type=jnp.float32) * a_out


def _tc_mid_body(p_ref, deg_ref, b_ref, w_ref, o_ref):
    a_out, a_in = _scales(deg_ref)
    t = (p_ref[0] + p_ref[1]) * a_in + b_ref[...]
    t = jnp.maximum(t, 0.0)
    o_ref[...] = jnp.dot(t, w_ref[...],
                         preferred_element_type=jnp.float32) * a_out


def _tc_final_body(p_ref, deg_ref, b_ref, o_ref):
    _, a_in = _scales(deg_ref)
    o_ref[...] = (p_ref[0] + p_ref[1]) * a_in + b_ref[...]


BLK = 1000
GRID = N // BLK

_deg_spec = pl.BlockSpec((NC, 2, BLK, 1), lambda b: (0, 0, b, 0))
_p_spec = pl.BlockSpec((NC, BLK, D), lambda b: (0, b, 0))
_w_spec = pl.BlockSpec((D, D), lambda b: (0, 0))
_b_spec = pl.BlockSpec((1, D), lambda b: (0, 0))
_x_spec = pl.BlockSpec((BLK, D), lambda b: (b, 0))

_tc_first = pl.pallas_call(
    _tc_first_body,
    grid=(GRID,),
    in_specs=[_x_spec, _deg_spec, _w_spec],
    out_specs=_x_spec,
    out_shape=jax.ShapeDtypeStruct((N, D), jnp.float32),
)

_tc_mid = pl.pallas_call(
    _tc_mid_body,
    grid=(GRID,),
    in_specs=[_p_spec, _deg_spec, _b_spec, _w_spec],
    out_specs=_x_spec,
    out_shape=jax.ShapeDtypeStruct((N, D), jnp.float32),
)

_tc_final = pl.pallas_call(
    _tc_final_body,
    grid=(GRID,),
    in_specs=[_p_spec, _deg_spec, _b_spec],
    out_specs=_x_spec,
    out_shape=jax.ShapeDtypeStruct((N, D), jnp.float32),
)


def kernel(x, edge_index, W1, b1, W2, b2, W3, b3):
    src = edge_index[0].reshape(NW, EPT)
    dst = edge_index[1].reshape(NW, EPT)
    pad_n = jnp.full((NW, EPT_PAD - EPT), N, jnp.int32)
    pad_ns = jnp.full((NW, EPT_S - EPT), N, jnp.int32)
    pad_0s = jnp.zeros((NW, EPT_S - EPT), jnp.int32)
    srcd = jnp.concatenate([src, pad_n], axis=1)
    dstd = jnp.concatenate([dst, pad_n], axis=1)
    pad_n2 = jnp.full((NW, EPT_PAD - EPT), N, jnp.int32)
    pad_02 = jnp.zeros((NW, EPT_PAD - EPT), jnp.int32)
    srcg = jnp.concatenate([src, pad_02], axis=1).reshape(NW, NCHUNK, CH)
    dstg = jnp.concatenate([dst, pad_n2], axis=1).reshape(NW, NCHUNK, CH)

    deg = _deg_kernel(srcd, dstd)
    deg4 = deg.reshape(NC, 2, NPAD, 1)
    b1r, b2r, b3r = (b.reshape(1, D) for b in (b1, b2, b3))

    h = _tc_first(x, deg4, W1)
    p = _spmm_kernel(h, srcg, dstg)
    h = _tc_mid(p, deg4, b1r, W2)
    p = _spmm_kernel(h, srcg, dstg)
    h = _tc_mid(p, deg4, b2r, W3)
    p = _spmm_kernel(h, srcg, dstg)
    return _tc_final(p, deg4, b3r)
